# baseline (speedup 1.0000x reference)
import functools

import jax
import jax.numpy as jnp
from jax import lax
from jax.experimental import pallas as pl
from jax.experimental.pallas import tpu as pltpu
from jax.experimental.pallas import tpu_sc as plsc

L = 3
K = 16
DIMS = 8
DIM_Z = 4
N = 2097152

NC = 2
NS = 16
NW = NC * NS
NSPLIT = 16
NQ = N // NSPLIT
ROWS_PER_W = NQ // NW
BLK = 2048
NBLK = ROWS_PER_W // BLK
G = BLK // 16
UNROLL = 4

CB_OFF = 0
CBT_OFF = CB_OFF + L * DIM_Z * K
P_OFF = CBT_OFF + L * DIM_Z * K
CNS_OFF = P_OFF + L * DIMS * K
WINS_OFF = CNS_OFF + L * K * 16
BINS_OFF = WINS_OFF + DIMS * DIM_Z * 16
PARAM_LEN = BINS_OFF + DIM_Z * 16


def _bf16_round(v):
    i = plsc.bitcast(v, jnp.int32)
    r = (i + jnp.int32(0x8000)) & jnp.int32(-65536)
    return plsc.bitcast(r, jnp.float32)


def _vq_body(x_hbm, par_hbm, out_hbm, xv0, xv1, ov0, ov1, parv,
             sin0, sin1, sout0, sout1):
    wid = lax.axis_index("s") * NC + lax.axis_index("c")

    pltpu.sync_copy(par_hbm, parv)

    lane8 = lax.iota(jnp.int32, 16) * DIMS
    U = UNROLL

    def make_group_body(xv, ov):
        def group_body(h, carry):
            cbtv = [[parv[pl.ds(CBT_OFF + (l * DIM_Z + j) * K, 16)]
                     for j in range(DIM_Z)] for l in range(L)]
            rowg = [jnp.zeros((16,), jnp.int32) + (h * U + p)
                    for p in range(U)]
            cold = [lane8 + d for d in range(DIMS)]
            xs = [[_bf16_round(plsc.load_gather(xv, [rowg[p], cold[d]]))
                   for d in range(DIMS)] for p in range(U)]

            z = [[] for _ in range(U)]
            for j in range(DIM_Z):
                binv = parv[pl.ds(BINS_OFF + j * 16, 16)]
                wv = [parv[pl.ds(WINS_OFF + (d * DIM_Z + j) * 16, 16)]
                      for d in range(DIMS)]
                for p in range(U):
                    acc = binv
                    for d in range(DIMS):
                        acc = acc + xs[p][d] * wv[d]
                    z[p].append(acc)

            r = [list(z[p]) for p in range(U)]
            bidx = [[] for _ in range(U)]
            for l in range(L):
                rt = [[_bf16_round(r[p][j]) for j in range(DIM_Z)]
                      for p in range(U)]
                best = [None] * U
                for k in range(K):
                    cnv = parv[pl.ds(CNS_OFF + (l * K + k) * 16, 16)]
                    cb_k = [cbtv[l][j][k] for j in range(DIM_Z)]
                    for p in range(U):
                        s = cnv
                        for j in range(DIM_Z):
                            s = s - rt[p][j] * cb_k[j]
                        si = plsc.bitcast(s, jnp.int32) & jnp.int32(-16)
                        if k:
                            si = si | jnp.int32(k)
                        sp = plsc.bitcast(si, jnp.float32)
                        best[p] = (sp if best[p] is None
                                   else jnp.minimum(best[p], sp))
                for p in range(U):
                    bi = plsc.bitcast(best[p], jnp.int32) & jnp.int32(15)
                    bidx[p].append(bi)
                    if l < L - 1:
                        q = [plsc.load_gather(
                                parv, [bi + (CB_OFF + (l * DIM_Z + j) * K)])
                             for j in range(DIM_Z)]
                        r[p] = [r[p][j] - q[j] for j in range(DIM_Z)]

            for p in range(U):
                for d in range(DIMS):
                    acc = plsc.load_gather(
                        parv, [bidx[p][0] + (P_OFF + d * K)])
                    for l in range(1, L):
                        acc = acc + plsc.load_gather(
                            parv, [bidx[p][l] + (P_OFF + (l * DIMS + d) * K)])
                    plsc.store_scatter(ov, [rowg[p], cold[d]], acc)
            return carry
        return group_body

    body0 = make_group_body(xv0, ov0)
    body1 = make_group_body(xv1, ov1)

    def rowslice(b):
        return pl.ds(pl.multiple_of((wid * ROWS_PER_W + b * BLK) // 16, G), G)

    def in_copy(b, xv, sem):
        return pltpu.make_async_copy(x_hbm.at[rowslice(b)], xv, sem)

    def out_copy(b, ov, sem):
        return pltpu.make_async_copy(ov, out_hbm.at[rowslice(b)], sem)

    in_copy(0, xv0, sin0).start()

    def pair_body(i, carry):
        b0 = i * 2
        b1 = b0 + 1
        in_copy(b1, xv1, sin1).start()
        in_copy(b0, xv0, sin0).wait()

        @pl.when(i > 0)
        def _():
            out_copy(b0, ov0, sout0).wait()
        lax.fori_loop(0, G // U, body0, 0, unroll=False)
        out_copy(b0, ov0, sout0).start()

        @pl.when(i < NBLK // 2 - 1)
        def _():
            in_copy(b0 + 2, xv0, sin0).start()
        in_copy(b1, xv1, sin1).wait()

        @pl.when(i > 0)
        def _():
            out_copy(b1, ov1, sout1).wait()
        lax.fori_loop(0, G // U, body1, 0, unroll=False)
        out_copy(b1, ov1, sout1).start()
        return carry

    lax.fori_loop(0, NBLK // 2, pair_body, 0, unroll=False)
    out_copy(NBLK - 2, ov0, sout0).wait()
    out_copy(NBLK - 1, ov1, sout1).wait()


@jax.jit
def _vq(x_flat, params):
    mesh = plsc.VectorSubcoreMesh(core_axis_name="c", subcore_axis_name="s")
    f = functools.partial(
        pl.kernel,
        mesh=mesh,
        out_type=jax.ShapeDtypeStruct((NQ // 16, 128), jnp.float32),
        scratch_types=[
            pltpu.VMEM((G, 128), jnp.float32),
            pltpu.VMEM((G, 128), jnp.float32),
            pltpu.VMEM((G, 128), jnp.float32),
            pltpu.VMEM((G, 128), jnp.float32),
            pltpu.VMEM((PARAM_LEN,), jnp.float32),
            pltpu.SemaphoreType.DMA,
            pltpu.SemaphoreType.DMA,
            pltpu.SemaphoreType.DMA,
            pltpu.SemaphoreType.DMA,
        ],
        compiler_params=pltpu.CompilerParams(needs_layout_passes=False),
    )(_vq_body)
    return f(x_flat, params)


def _bf16_round_host(a):
    i = lax.bitcast_convert_type(a, jnp.int32)
    r = (i + jnp.int32(0x7FFF) + ((i >> 16) & jnp.int32(1))) & jnp.int32(-65536)
    return lax.bitcast_convert_type(r, jnp.float32)


def kernel(x, codebooks, W_in, b_in, W_out, b_out):
    cb_t = jnp.transpose(codebooks, (0, 2, 1))
    cbt_bf = _bf16_round_host(cb_t)
    cn = 0.5 * jnp.sum(codebooks * codebooks, axis=-1)
    win_bf = _bf16_round_host(W_in)
    pcb = jnp.einsum("lkj,jd->ldk", codebooks, _bf16_round_host(W_out))
    pcb = pcb.at[0].add(b_out[:, None])
    cns = jnp.broadcast_to(cn.reshape(L * K, 1), (L * K, 16))
    wins = jnp.broadcast_to(win_bf.reshape(DIMS * DIM_Z, 1),
                            (DIMS * DIM_Z, 16))
    bins = jnp.broadcast_to(b_in.reshape(DIM_Z, 1), (DIM_Z, 16))
    params = jnp.concatenate([
        cb_t.reshape(-1),
        cbt_bf.reshape(-1),
        pcb.reshape(-1),
        cns.reshape(-1),
        wins.reshape(-1),
        bins.reshape(-1),
    ])
    x3 = x.reshape(N // 128, 128, DIMS)
    ys = []
    for i in range(NSPLIT):
        xi = lax.slice_in_dim(x3, i * (NQ // 128), (i + 1) * (NQ // 128),
                              axis=0)
        yi = _vq(xi.reshape(NQ // 16, 128), params)
        ys.append(yi.reshape(NQ, DIMS))
    return jnp.concatenate(ys, axis=0)


# speedup vs baseline: 1.2901x; 1.2901x over previous
import functools

import jax
import jax.numpy as jnp
from jax import lax
from jax.experimental import pallas as pl
from jax.experimental.pallas import tpu as pltpu
from jax.experimental.pallas import tpu_sc as plsc

L = 3
K = 16
DIMS = 8
DIM_Z = 4
N = 2097152

NC = 2
NS = 16
NW = NC * NS
NSPLIT = 8
NQ = N // NSPLIT
ROWS_PER_W = NQ // NW
BLK = 2048
NBLK = ROWS_PER_W // BLK
G = BLK // 16
UNROLL = 4

CB_OFF = 0
CBT_OFF = CB_OFF + L * DIM_Z * K
P_OFF = CBT_OFF + L * DIM_Z * K
CNS_OFF = P_OFF + L * DIMS * K
WINS_OFF = CNS_OFF + L * K * 16
BINS_OFF = WINS_OFF + DIMS * DIM_Z * 16
PARAM_LEN = BINS_OFF + DIM_Z * 16


def _bf16_round(v):
    i = plsc.bitcast(v, jnp.int32)
    r = (i + jnp.int32(0x8000)) & jnp.int32(-65536)
    return plsc.bitcast(r, jnp.float32)


def _vq_body(x_hbm, par_hbm, out_hbm, xv0, xv1, ov0, ov1, parv,
             sin0, sin1, sout0, sout1):
    wid = lax.axis_index("s") * NC + lax.axis_index("c")

    pltpu.sync_copy(par_hbm, parv)

    lane8 = lax.iota(jnp.int32, 16) * DIMS
    U = UNROLL

    def make_group_body(xv, ov):
        def group_body(h, carry):
            cbtv = [[parv[pl.ds(CBT_OFF + (l * DIM_Z + j) * K, 16)]
                     for j in range(DIM_Z)] for l in range(L)]
            rowg = [jnp.zeros((16,), jnp.int32) + (h * U + p)
                    for p in range(U)]
            cold = [lane8 + d for d in range(DIMS)]
            xs = [[_bf16_round(plsc.load_gather(xv, [rowg[p], cold[d]]))
                   for d in range(DIMS)] for p in range(U)]

            z = [[] for _ in range(U)]
            for j in range(DIM_Z):
                binv = parv[pl.ds(BINS_OFF + j * 16, 16)]
                wv = [parv[pl.ds(WINS_OFF + (d * DIM_Z + j) * 16, 16)]
                      for d in range(DIMS)]
                for p in range(U):
                    acc = binv
                    for d in range(DIMS):
                        acc = acc + xs[p][d] * wv[d]
                    z[p].append(acc)

            r = [list(z[p]) for p in range(U)]
            bidx = [[] for _ in range(U)]
            for l in range(L):
                rt = [[_bf16_round(r[p][j]) for j in range(DIM_Z)]
                      for p in range(U)]
                best = [None] * U
                for k in range(K):
                    cnv = parv[pl.ds(CNS_OFF + (l * K + k) * 16, 16)]
                    cb_k = [cbtv[l][j][k] for j in range(DIM_Z)]
                    for p in range(U):
                        s = cnv
                        for j in range(DIM_Z):
                            s = s - rt[p][j] * cb_k[j]
                        si = plsc.bitcast(s, jnp.int32) & jnp.int32(-16)
                        if k:
                            si = si | jnp.int32(k)
                        sp = plsc.bitcast(si, jnp.float32)
                        best[p] = (sp if best[p] is None
                                   else jnp.minimum(best[p], sp))
                for p in range(U):
                    bi = plsc.bitcast(best[p], jnp.int32) & jnp.int32(15)
                    bidx[p].append(bi)
                    if l < L - 1:
                        q = [plsc.load_gather(
                                parv, [bi + (CB_OFF + (l * DIM_Z + j) * K)])
                             for j in range(DIM_Z)]
                        r[p] = [r[p][j] - q[j] for j in range(DIM_Z)]

            for p in range(U):
                for d in range(DIMS):
                    acc = plsc.load_gather(
                        parv, [bidx[p][0] + (P_OFF + d * K)])
                    for l in range(1, L):
                        acc = acc + plsc.load_gather(
                            parv, [bidx[p][l] + (P_OFF + (l * DIMS + d) * K)])
                    plsc.store_scatter(ov, [rowg[p], cold[d]], acc)
            return carry
        return group_body

    body0 = make_group_body(xv0, ov0)
    body1 = make_group_body(xv1, ov1)

    def rowslice(b):
        return pl.ds(pl.multiple_of((wid * ROWS_PER_W + b * BLK) // 16, G), G)

    def in_copy(b, xv, sem):
        return pltpu.make_async_copy(x_hbm.at[rowslice(b)], xv, sem)

    def out_copy(b, ov, sem):
        return pltpu.make_async_copy(ov, out_hbm.at[rowslice(b)], sem)

    in_copy(0, xv0, sin0).start()

    def pair_body(i, carry):
        b0 = i * 2
        b1 = b0 + 1
        in_copy(b1, xv1, sin1).start()
        in_copy(b0, xv0, sin0).wait()

        @pl.when(i > 0)
        def _():
            out_copy(b0, ov0, sout0).wait()
        lax.fori_loop(0, G // U, body0, 0, unroll=False)
        out_copy(b0, ov0, sout0).start()

        @pl.when(i < NBLK // 2 - 1)
        def _():
            in_copy(b0 + 2, xv0, sin0).start()
        in_copy(b1, xv1, sin1).wait()

        @pl.when(i > 0)
        def _():
            out_copy(b1, ov1, sout1).wait()
        lax.fori_loop(0, G // U, body1, 0, unroll=False)
        out_copy(b1, ov1, sout1).start()
        return carry

    lax.fori_loop(0, NBLK // 2, pair_body, 0, unroll=False)
    out_copy(NBLK - 2, ov0, sout0).wait()
    out_copy(NBLK - 1, ov1, sout1).wait()


@jax.jit
def _vq(x_flat, params):
    mesh = plsc.VectorSubcoreMesh(core_axis_name="c", subcore_axis_name="s")
    f = functools.partial(
        pl.kernel,
        mesh=mesh,
        out_type=jax.ShapeDtypeStruct((NQ // 16, 128), jnp.float32),
        scratch_types=[
            pltpu.VMEM((G, 128), jnp.float32),
            pltpu.VMEM((G, 128), jnp.float32),
            pltpu.VMEM((G, 128), jnp.float32),
            pltpu.VMEM((G, 128), jnp.float32),
            pltpu.VMEM((PARAM_LEN,), jnp.float32),
            pltpu.SemaphoreType.DMA,
            pltpu.SemaphoreType.DMA,
            pltpu.SemaphoreType.DMA,
            pltpu.SemaphoreType.DMA,
        ],
        compiler_params=pltpu.CompilerParams(needs_layout_passes=False),
    )(_vq_body)
    return f(x_flat, params)


def _bf16_round_host(a):
    i = lax.bitcast_convert_type(a, jnp.int32)
    r = (i + jnp.int32(0x7FFF) + ((i >> 16) & jnp.int32(1))) & jnp.int32(-65536)
    return lax.bitcast_convert_type(r, jnp.float32)


def kernel(x, codebooks, W_in, b_in, W_out, b_out):
    cb_t = jnp.transpose(codebooks, (0, 2, 1))
    cbt_bf = _bf16_round_host(cb_t)
    cn = 0.5 * jnp.sum(codebooks * codebooks, axis=-1)
    win_bf = _bf16_round_host(W_in)
    pcb = jnp.einsum("lkj,jd->ldk", codebooks, _bf16_round_host(W_out))
    pcb = pcb.at[0].add(b_out[:, None])
    cns = jnp.broadcast_to(cn.reshape(L * K, 1), (L * K, 16))
    wins = jnp.broadcast_to(win_bf.reshape(DIMS * DIM_Z, 1),
                            (DIMS * DIM_Z, 16))
    bins = jnp.broadcast_to(b_in.reshape(DIM_Z, 1), (DIM_Z, 16))
    params = jnp.concatenate([
        cb_t.reshape(-1),
        cbt_bf.reshape(-1),
        pcb.reshape(-1),
        cns.reshape(-1),
        wins.reshape(-1),
        bins.reshape(-1),
    ])
    x3 = x.reshape(N // 128, 128, DIMS)
    ys = []
    for i in range(NSPLIT):
        xi = lax.slice_in_dim(x3, i * (NQ // 128), (i + 1) * (NQ // 128),
                              axis=0)
        yi = _vq(xi.reshape(NQ // 16, 128), params)
        ys.append(yi.reshape(NQ, DIMS))
    return jnp.concatenate(ys, axis=0)
